# Initial kernel scaffold; baseline (speedup 1.0000x reference)
import functools

import jax
import jax.numpy as jnp
from jax import lax
from jax.experimental import pallas as pl
from jax.experimental.pallas import tpu as pltpu
from jax.experimental.pallas import tpu_sc as plsc

N = 10000
E = 320000
F = 128
OP = 16
NC, NS = 2, 16
NW = NC * NS
WIN = 128
NWIN = 79
EPT = NWIN * WIN
EP = NW * EPT
GR = 240
NR = N + GR
STR = NR // NS
BLK = 256
GRID = NR // BLK


def _sc_mesh():
    return plsc.VectorSubcoreMesh(core_axis_name="c", subcore_axis_name="s")


@functools.partial(
    pl.kernel,
    out_type=(
        jax.ShapeDtypeStruct((NC, NR, OP), jnp.float32),
        jax.ShapeDtypeStruct((NC, NR, OP), jnp.float32),
    ),
    mesh=_sc_mesh(),
    scratch_types=(
        pltpu.VMEM((NWIN, WIN), jnp.int32),
        pltpu.VMEM((NWIN, WIN), jnp.int32),
        pltpu.VMEM((WIN, OP), jnp.float32),
        pltpu.VMEM_SHARED((NR, OP), jnp.float32),
        pltpu.VMEM_SHARED((NR, OP), jnp.float32),
    ),
)
def _sc_degrees(src_hbm, dst_hbm, ones_hbm, zeros_hbm, do_out, di_out,
                idx_s, idx_d, ones_v, sh_do, sh_di):
    c = lax.axis_index("c")
    s = lax.axis_index("s")
    wid = s * NC + c
    pltpu.sync_copy(src_hbm.at[wid], idx_s)
    pltpu.sync_copy(dst_hbm.at[wid], idx_d)
    pltpu.sync_copy(ones_hbm, ones_v)
    pltpu.sync_copy(zeros_hbm, sh_do.at[pl.ds(s * STR, STR)])
    pltpu.sync_copy(zeros_hbm, sh_di.at[pl.ds(s * STR, STR)])
    plsc.subcore_barrier()

    @pl.loop(0, NWIN)
    def _(w):
        pltpu.sync_copy(ones_v, sh_do.at[idx_s.at[w]], add=True)
        pltpu.sync_copy(ones_v, sh_di.at[idx_d.at[w]], add=True)

    plsc.subcore_barrier()
    pltpu.sync_copy(sh_do.at[pl.ds(s * STR, STR)],
                    do_out.at[c, pl.ds(s * STR, STR)])
    pltpu.sync_copy(sh_di.at[pl.ds(s * STR, STR)],
                    di_out.at[c, pl.ds(s * STR, STR)])


def _make_sc_agg(d):
    @functools.partial(
        pl.kernel,
        out_type=jax.ShapeDtypeStruct((NC, NR, d), jnp.float32),
        mesh=_sc_mesh(),
        scratch_types=(
            pltpu.VMEM((NWIN, WIN), jnp.int32),
            pltpu.VMEM((NWIN, WIN), jnp.int32),
            pltpu.VMEM((WIN, d), jnp.float32),
            pltpu.VMEM_SHARED((NR, d), jnp.float32),
            pltpu.SemaphoreType.DMA,
        ),
    )
    def _sc_agg(rows_hbm, src_hbm, dst_hbm, zeros_hbm, out,
                idx_s, idx_d, rows_v, sh, sem):
        c = lax.axis_index("c")
        s = lax.axis_index("s")
        wid = s * NC + c
        pltpu.sync_copy(src_hbm.at[wid], idx_s)
        pltpu.sync_copy(dst_hbm.at[wid], idx_d)
        pltpu.sync_copy(zeros_hbm, sh.at[pl.ds(s * STR, STR)])
        plsc.subcore_barrier()

        @pl.loop(0, NWIN)
        def _(w):
            pltpu.async_copy(rows_hbm.at[idx_s.at[w]], rows_v, sem).wait()
            pltpu.sync_copy(rows_v, sh.at[idx_d.at[w]], add=True)

        plsc.subcore_barrier()
        pltpu.sync_copy(sh.at[pl.ds(s * STR, STR)],
                        out.at[c, pl.ds(s * STR, STR)])

    return _sc_agg


_sc_agg128 = _make_sc_agg(F)
_sc_agg16 = _make_sc_agg(OP)


def _tc_norms_body(x_ref, do_ref, di_ref, xs_ref, ns_ref, nd_ref):
    ns = lax.rsqrt(jnp.maximum(do_ref[0] + do_ref[1], 1.0))
    nd = lax.rsqrt(jnp.maximum(di_ref[0] + di_ref[1], 1.0))
    ns_ref[...] = ns
    nd_ref[...] = nd
    xs_ref[...] = x_ref[...] * ns[:, :1]


def _tc_norms(x_pad, do_p, di_p):
    return pl.pallas_call(
        _tc_norms_body,
        grid=(GRID,),
        in_specs=[
            pl.BlockSpec((BLK, F), lambda i: (i, 0)),
            pl.BlockSpec((NC, BLK, OP), lambda i: (0, i, 0)),
            pl.BlockSpec((NC, BLK, OP), lambda i: (0, i, 0)),
        ],
        out_specs=[
            pl.BlockSpec((BLK, F), lambda i: (i, 0)),
            pl.BlockSpec((BLK, OP), lambda i: (i, 0)),
            pl.BlockSpec((BLK, OP), lambda i: (i, 0)),
        ],
        out_shape=[
            jax.ShapeDtypeStruct((NR, F), jnp.float32),
            jax.ShapeDtypeStruct((NR, OP), jnp.float32),
            jax.ShapeDtypeStruct((NR, OP), jnp.float32),
        ],
    )(x_pad, do_p, di_p)


def _tc_mid_body(agg_ref, nd_ref, w1_ref, b1_ref, w2_ref, ns_ref, out_ref):
    t = (agg_ref[0] + agg_ref[1]) * nd_ref[:, :1]
    x1 = jnp.dot(t, w1_ref[...], preferred_element_type=jnp.float32)
    x1 = jnp.maximum(x1 + b1_ref[...], 0.0)
    h2 = jnp.dot(x1, w2_ref[...], preferred_element_type=jnp.float32)
    out_ref[...] = h2 * ns_ref[:, :1]


def _tc_mid(agg_p, nd, w1, b1_2d, w2p, ns):
    return pl.pallas_call(
        _tc_mid_body,
        grid=(GRID,),
        in_specs=[
            pl.BlockSpec((NC, BLK, F), lambda i: (0, i, 0)),
            pl.BlockSpec((BLK, OP), lambda i: (i, 0)),
            pl.BlockSpec((F, F), lambda i: (0, 0)),
            pl.BlockSpec((1, F), lambda i: (0, 0)),
            pl.BlockSpec((F, OP), lambda i: (0, 0)),
            pl.BlockSpec((BLK, OP), lambda i: (i, 0)),
        ],
        out_specs=pl.BlockSpec((BLK, OP), lambda i: (i, 0)),
        out_shape=jax.ShapeDtypeStruct((NR, OP), jnp.float32),
    )(agg_p, nd, w1, b1_2d, w2p, ns)


def _tc_final_body(agg_ref, nd_ref, b2_ref, z_ref, att_ref):
    t = (agg_ref[0] + agg_ref[1]) * nd_ref[:, :1] + b2_ref[...]
    x2 = jnp.maximum(t, 0.0)
    zv = jnp.sum(x2, axis=1, keepdims=True) * (1.0 / 5.0)
    z_ref[...] = jnp.broadcast_to(zv, (BLK, OP))
    att_ref[...] = jnp.ones((BLK, OP), jnp.float32)


def _tc_final(agg2_p, nd, b2p_2d):
    return pl.pallas_call(
        _tc_final_body,
        grid=(GRID,),
        in_specs=[
            pl.BlockSpec((NC, BLK, OP), lambda i: (0, i, 0)),
            pl.BlockSpec((BLK, OP), lambda i: (i, 0)),
            pl.BlockSpec((1, OP), lambda i: (0, 0)),
        ],
        out_specs=[
            pl.BlockSpec((BLK, OP), lambda i: (i, 0)),
            pl.BlockSpec((BLK, OP), lambda i: (i, 0)),
        ],
        out_shape=[
            jax.ShapeDtypeStruct((NR, OP), jnp.float32),
            jax.ShapeDtypeStruct((NR, OP), jnp.float32),
        ],
    )(agg2_p, nd, b2p_2d)


def kernel(features, edge_index, W1, b1, W2, b2, W_att, b_att):
    n_nodes = features.shape[-1]
    x = jnp.reshape(features, (n_nodes, -1))
    x_pad = jnp.pad(x, ((0, NR - N), (0, 0)))

    pad_ids = (N + (jnp.arange(EP - E, dtype=jnp.int32) % GR))
    src = jnp.concatenate([edge_index[0].astype(jnp.int32), pad_ids])
    dst = jnp.concatenate([edge_index[1].astype(jnp.int32), pad_ids])
    src = src.reshape(NW, NWIN, WIN)
    dst = dst.reshape(NW, NWIN, WIN)

    ones_w = jnp.ones((WIN, OP), jnp.float32)
    zeros_s16 = jnp.zeros((STR, OP), jnp.float32)
    zeros_s128 = jnp.zeros((STR, F), jnp.float32)

    do_p, di_p = _sc_degrees(src, dst, ones_w, zeros_s16)
    xs, ns, nd = _tc_norms(x_pad, do_p, di_p)
    agg1_p = _sc_agg128(xs, src, dst, zeros_s128)

    b1_2d = jnp.reshape(b1, (1, F))
    w2p = jnp.pad(W2, ((0, 0), (0, OP - W2.shape[1])))
    h2s = _tc_mid(agg1_p, nd, W1, b1_2d, w2p, ns)

    agg2_p = _sc_agg16(h2s, src, dst, zeros_s16)

    b2p_2d = jnp.reshape(jnp.pad(b2, (0, OP - b2.shape[0])), (1, OP))
    z16, att16 = _tc_final(agg2_p, nd, b2p_2d)

    z = z16[:N, 0]
    att = att16[:N, :1]
    return (z, att)


# speedup vs baseline: 12.6911x; 12.6911x over previous
import functools

import jax
import jax.numpy as jnp
from jax import lax
from jax.experimental import pallas as pl
from jax.experimental.pallas import tpu as pltpu
from jax.experimental.pallas import tpu_sc as plsc

N = 10000
E = 320000
F = 128
OP = 16
NC, NS = 2, 16
NW = NC * NS
WIN = 128
NWIN = 79
EPT = NWIN * WIN
EP = NW * EPT
GR = 240
NR = N + GR
STR = NR // NS
BLK = 256
GRID = NR // BLK


def _sc_mesh():
    return plsc.VectorSubcoreMesh(core_axis_name="c", subcore_axis_name="s")


@functools.partial(
    pl.kernel,
    out_type=(
        jax.ShapeDtypeStruct((NC, NR, OP), jnp.float32),
        jax.ShapeDtypeStruct((NC, NR, OP), jnp.float32),
    ),
    mesh=_sc_mesh(),
    compiler_params=pltpu.CompilerParams(use_tc_tiling_on_sc=False),
    scratch_types=(
        pltpu.VMEM((NWIN, WIN), jnp.int32),
        pltpu.VMEM((NWIN, WIN), jnp.int32),
        pltpu.VMEM((WIN, OP), jnp.float32),
        pltpu.VMEM_SHARED((NR, OP), jnp.float32),
        pltpu.VMEM_SHARED((NR, OP), jnp.float32),
    ),
)
def _sc_degrees(src_hbm, dst_hbm, ones_hbm, zeros_hbm, do_out, di_out,
                idx_s, idx_d, ones_v, sh_do, sh_di):
    c = lax.axis_index("c")
    s = lax.axis_index("s")
    wid = s * NC + c
    pltpu.sync_copy(src_hbm.at[wid], idx_s)
    pltpu.sync_copy(dst_hbm.at[wid], idx_d)
    pltpu.sync_copy(ones_hbm, ones_v)
    pltpu.sync_copy(zeros_hbm, sh_do.at[pl.ds(s * STR, STR)])
    pltpu.sync_copy(zeros_hbm, sh_di.at[pl.ds(s * STR, STR)])
    plsc.subcore_barrier()

    @pl.loop(0, NWIN)
    def _(w):
        pltpu.sync_copy(ones_v, sh_do.at[idx_s.at[w]], add=True)
        pltpu.sync_copy(ones_v, sh_di.at[idx_d.at[w]], add=True)

    plsc.subcore_barrier()
    pltpu.sync_copy(sh_do.at[pl.ds(s * STR, STR)],
                    do_out.at[c, pl.ds(s * STR, STR)])
    pltpu.sync_copy(sh_di.at[pl.ds(s * STR, STR)],
                    di_out.at[c, pl.ds(s * STR, STR)])


def _make_sc_agg(d):
    @functools.partial(
        pl.kernel,
        out_type=jax.ShapeDtypeStruct((NC, NR, d), jnp.float32),
        mesh=_sc_mesh(),
        compiler_params=pltpu.CompilerParams(use_tc_tiling_on_sc=False),
        scratch_types=(
            pltpu.VMEM((NWIN, WIN), jnp.int32),
            pltpu.VMEM((NWIN, WIN), jnp.int32),
            pltpu.VMEM((WIN, d), jnp.float32),
            pltpu.VMEM_SHARED((NR, d), jnp.float32),
            pltpu.SemaphoreType.DMA,
        ),
    )
    def _sc_agg(rows_hbm, src_hbm, dst_hbm, zeros_hbm, out,
                idx_s, idx_d, rows_v, sh, sem):
        c = lax.axis_index("c")
        s = lax.axis_index("s")
        wid = s * NC + c
        pltpu.sync_copy(src_hbm.at[wid], idx_s)
        pltpu.sync_copy(dst_hbm.at[wid], idx_d)
        pltpu.sync_copy(zeros_hbm, sh.at[pl.ds(s * STR, STR)])
        plsc.subcore_barrier()

        @pl.loop(0, NWIN)
        def _(w):
            pltpu.async_copy(rows_hbm.at[idx_s.at[w]], rows_v, sem).wait()
            pltpu.sync_copy(rows_v, sh.at[idx_d.at[w]], add=True)

        plsc.subcore_barrier()
        pltpu.sync_copy(sh.at[pl.ds(s * STR, STR)],
                        out.at[c, pl.ds(s * STR, STR)])

    return _sc_agg


_sc_agg128 = _make_sc_agg(F)
_sc_agg16 = _make_sc_agg(OP)


def _tc_norms_body(x_ref, do_ref, di_ref, xs_ref, ns_ref, nd_ref):
    ns = lax.rsqrt(jnp.maximum(do_ref[0] + do_ref[1], 1.0))
    nd = lax.rsqrt(jnp.maximum(di_ref[0] + di_ref[1], 1.0))
    ns_ref[...] = ns
    nd_ref[...] = nd
    xs_ref[...] = x_ref[...] * ns[:, :1]


def _tc_norms(x_pad, do_p, di_p):
    return pl.pallas_call(
        _tc_norms_body,
        grid=(GRID,),
        in_specs=[
            pl.BlockSpec((BLK, F), lambda i: (i, 0)),
            pl.BlockSpec((NC, BLK, OP), lambda i: (0, i, 0)),
            pl.BlockSpec((NC, BLK, OP), lambda i: (0, i, 0)),
        ],
        out_specs=[
            pl.BlockSpec((BLK, F), lambda i: (i, 0)),
            pl.BlockSpec((BLK, OP), lambda i: (i, 0)),
            pl.BlockSpec((BLK, OP), lambda i: (i, 0)),
        ],
        out_shape=[
            jax.ShapeDtypeStruct((NR, F), jnp.float32),
            jax.ShapeDtypeStruct((NR, OP), jnp.float32),
            jax.ShapeDtypeStruct((NR, OP), jnp.float32),
        ],
    )(x_pad, do_p, di_p)


def _tc_mid_body(agg_ref, nd_ref, w1_ref, b1_ref, w2_ref, ns_ref, out_ref):
    t = (agg_ref[0] + agg_ref[1]) * nd_ref[:, :1]
    x1 = jnp.dot(t, w1_ref[...], preferred_element_type=jnp.float32)
    x1 = jnp.maximum(x1 + b1_ref[...], 0.0)
    h2 = jnp.dot(x1, w2_ref[...], preferred_element_type=jnp.float32)
    out_ref[...] = h2 * ns_ref[:, :1]


def _tc_mid(agg_p, nd, w1, b1_2d, w2p, ns):
    return pl.pallas_call(
        _tc_mid_body,
        grid=(GRID,),
        in_specs=[
            pl.BlockSpec((NC, BLK, F), lambda i: (0, i, 0)),
            pl.BlockSpec((BLK, OP), lambda i: (i, 0)),
            pl.BlockSpec((F, F), lambda i: (0, 0)),
            pl.BlockSpec((1, F), lambda i: (0, 0)),
            pl.BlockSpec((F, OP), lambda i: (0, 0)),
            pl.BlockSpec((BLK, OP), lambda i: (i, 0)),
        ],
        out_specs=pl.BlockSpec((BLK, OP), lambda i: (i, 0)),
        out_shape=jax.ShapeDtypeStruct((NR, OP), jnp.float32),
    )(agg_p, nd, w1, b1_2d, w2p, ns)


def _tc_final_body(agg_ref, nd_ref, b2_ref, z_ref, att_ref):
    t = (agg_ref[0] + agg_ref[1]) * nd_ref[:, :1] + b2_ref[...]
    x2 = jnp.maximum(t, 0.0)
    zv = jnp.sum(x2, axis=1, keepdims=True) * (1.0 / 5.0)
    z_ref[...] = jnp.broadcast_to(zv, (BLK, OP))
    att_ref[...] = jnp.ones((BLK, OP), jnp.float32)


def _tc_final(agg2_p, nd, b2p_2d):
    return pl.pallas_call(
        _tc_final_body,
        grid=(GRID,),
        in_specs=[
            pl.BlockSpec((NC, BLK, OP), lambda i: (0, i, 0)),
            pl.BlockSpec((BLK, OP), lambda i: (i, 0)),
            pl.BlockSpec((1, OP), lambda i: (0, 0)),
        ],
        out_specs=[
            pl.BlockSpec((BLK, OP), lambda i: (i, 0)),
            pl.BlockSpec((BLK, OP), lambda i: (i, 0)),
        ],
        out_shape=[
            jax.ShapeDtypeStruct((NR, OP), jnp.float32),
            jax.ShapeDtypeStruct((NR, OP), jnp.float32),
        ],
    )(agg2_p, nd, b2p_2d)


def kernel(features, edge_index, W1, b1, W2, b2, W_att, b_att):
    n_nodes = features.shape[-1]
    x = jnp.reshape(features, (n_nodes, -1))
    x_pad = jnp.pad(x, ((0, NR - N), (0, 0)))

    pad_ids = (N + (jnp.arange(EP - E, dtype=jnp.int32) % GR))
    src = jnp.concatenate([edge_index[0].astype(jnp.int32), pad_ids])
    dst = jnp.concatenate([edge_index[1].astype(jnp.int32), pad_ids])
    src = src.reshape(NW, NWIN, WIN)
    dst = dst.reshape(NW, NWIN, WIN)

    ones_w = jnp.ones((WIN, OP), jnp.float32)
    zeros_s16 = jnp.zeros((STR, OP), jnp.float32)
    zeros_s128 = jnp.zeros((STR, F), jnp.float32)

    do_p, di_p = _sc_degrees(src, dst, ones_w, zeros_s16)
    xs, ns, nd = _tc_norms(x_pad, do_p, di_p)
    agg1_p = _sc_agg128(xs, src, dst, zeros_s128)

    b1_2d = jnp.reshape(b1, (1, F))
    w2p = jnp.pad(W2, ((0, 0), (0, OP - W2.shape[1])))
    h2s = _tc_mid(agg1_p, nd, W1, b1_2d, w2p, ns)

    agg2_p = _sc_agg16(h2s, src, dst, zeros_s16)

    b2p_2d = jnp.reshape(jnp.pad(b2, (0, OP - b2.shape[0])), (1, OP))
    z16, att16 = _tc_final(agg2_p, nd, b2p_2d)

    z = z16[:N, 0]
    att = att16[:N, :1]
    return (z, att)


# speedup vs baseline: 13.9511x; 1.0993x over previous
import functools

import jax
import jax.numpy as jnp
from jax import lax
from jax.experimental import pallas as pl
from jax.experimental.pallas import tpu as pltpu
from jax.experimental.pallas import tpu_sc as plsc

N = 10000
E = 320000
F = 128
FH = F // 2
OP = 16
NC, NS = 2, 16
WIN = 128
NWIN = 160
NWH = NWIN // 2
EPT = NWIN * WIN
EP = NS * EPT
GR = 240
NR = N + GR
STR = NR // NS
CH1 = 10
CH3 = 8
BLK = 256
GRID = NR // BLK

_SC_PARAMS = pltpu.CompilerParams(use_tc_tiling_on_sc=False)


def _sc_mesh():
    return plsc.VectorSubcoreMesh(core_axis_name="c", subcore_axis_name="s")


@functools.partial(
    pl.kernel,
    out_type=(
        jax.ShapeDtypeStruct((NC, NR, OP), jnp.float32),
        jax.ShapeDtypeStruct((NC, NR, OP), jnp.float32),
    ),
    mesh=_sc_mesh(),
    compiler_params=_SC_PARAMS,
    scratch_types=(
        pltpu.VMEM((NWH, WIN), jnp.int32),
        pltpu.VMEM((NWH, WIN), jnp.int32),
        pltpu.VMEM((WIN, OP), jnp.float32),
        pltpu.VMEM_SHARED((NR, OP), jnp.float32),
        pltpu.VMEM_SHARED((NR, OP), jnp.float32),
        pltpu.SemaphoreType.DMA,
    ),
)
def _sc_degrees(src_hbm, dst_hbm, ones_hbm, zeros_hbm, do_out, di_out,
                idx_s, idx_d, ones_v, sh_do, sh_di, sem):
    c = lax.axis_index("c")
    s = lax.axis_index("s")
    pltpu.sync_copy(src_hbm.at[s, pl.ds(c * NWH, NWH)], idx_s)
    pltpu.sync_copy(dst_hbm.at[s, pl.ds(c * NWH, NWH)], idx_d)
    pltpu.sync_copy(ones_hbm, ones_v)
    pltpu.sync_copy(zeros_hbm, sh_do.at[pl.ds(s * STR, STR)])
    pltpu.sync_copy(zeros_hbm, sh_di.at[pl.ds(s * STR, STR)])
    plsc.subcore_barrier()

    @pl.loop(0, NWH // CH1)
    def _(k):
        for j in range(CH1):
            w = k * CH1 + j
            pltpu.async_copy(ones_v, sh_do.at[idx_s.at[w]], sem, add=True)
            pltpu.async_copy(ones_v, sh_di.at[idx_d.at[w]], sem, add=True)
        for j in range(CH1):
            w = k * CH1 + j
            pltpu.make_async_copy(ones_v, sh_do.at[idx_s.at[w]], sem).wait()
            pltpu.make_async_copy(ones_v, sh_di.at[idx_d.at[w]], sem).wait()

    plsc.subcore_barrier()
    pltpu.sync_copy(sh_do.at[pl.ds(s * STR, STR)],
                    do_out.at[c, pl.ds(s * STR, STR)])
    pltpu.sync_copy(sh_di.at[pl.ds(s * STR, STR)],
                    di_out.at[c, pl.ds(s * STR, STR)])


@functools.partial(
    pl.kernel,
    out_type=jax.ShapeDtypeStruct((NC, NR, FH), jnp.float32),
    mesh=_sc_mesh(),
    compiler_params=_SC_PARAMS,
    scratch_types=(
        pltpu.VMEM((NWIN, WIN), jnp.int32),
        pltpu.VMEM((NWIN, WIN), jnp.int32),
        pltpu.VMEM((WIN, FH), jnp.float32),
        pltpu.VMEM((WIN, FH), jnp.float32),
        pltpu.VMEM_SHARED((NR, FH), jnp.float32),
        pltpu.SemaphoreType.DMA,
        pltpu.SemaphoreType.DMA,
    ),
)
def _sc_agg128(xs0_hbm, xs1_hbm, src_hbm, dst_hbm, zeros_hbm, out,
               idx_s, idx_d, r0, r1, sh, sem0, sem1):
    c = lax.axis_index("c")
    s = lax.axis_index("s")
    pltpu.sync_copy(src_hbm.at[s], idx_s)
    pltpu.sync_copy(dst_hbm.at[s], idx_d)
    pltpu.sync_copy(zeros_hbm, sh.at[pl.ds(s * STR, STR)])
    plsc.subcore_barrier()

    def run_half(xs_ref):
        pltpu.async_copy(xs_ref.at[idx_s.at[0]], r0, sem0)

        @pl.loop(0, NWIN // 2)
        def _(p):
            w = 2 * p
            pltpu.make_async_copy(xs_ref.at[idx_s.at[w]], r0, sem0).wait()
            pltpu.async_copy(xs_ref.at[idx_s.at[w + 1]], r1, sem1)
            pltpu.sync_copy(r0, sh.at[idx_d.at[w]], add=True)
            pltpu.make_async_copy(xs_ref.at[idx_s.at[w + 1]], r1, sem1).wait()

            @pl.when(w + 2 < NWIN)
            def _():
                pltpu.async_copy(xs_ref.at[idx_s.at[w + 2]], r0, sem0)

            pltpu.sync_copy(r1, sh.at[idx_d.at[w + 1]], add=True)

    @pl.when(c == 0)
    def _():
        run_half(xs0_hbm)

    @pl.when(c == 1)
    def _():
        run_half(xs1_hbm)

    plsc.subcore_barrier()
    pltpu.sync_copy(sh.at[pl.ds(s * STR, STR)],
                    out.at[c, pl.ds(s * STR, STR)])


@functools.partial(
    pl.kernel,
    out_type=jax.ShapeDtypeStruct((NC, NR, OP), jnp.float32),
    mesh=_sc_mesh(),
    compiler_params=_SC_PARAMS,
    scratch_types=(
        pltpu.VMEM((NWH, WIN), jnp.int32),
        pltpu.VMEM((NWH, WIN), jnp.int32),
        pltpu.VMEM((CH3, WIN, OP), jnp.float32),
        pltpu.VMEM((CH3, WIN, OP), jnp.float32),
        pltpu.VMEM_SHARED((NR, OP), jnp.float32),
        pltpu.SemaphoreType.DMA,
        pltpu.SemaphoreType.DMA,
    ),
)
def _sc_agg16(rows_hbm, src_hbm, dst_hbm, zeros_hbm, out,
              idx_s, idx_d, r0, r1, sh, sem0, sem1):
    c = lax.axis_index("c")
    s = lax.axis_index("s")
    pltpu.sync_copy(src_hbm.at[s, pl.ds(c * NWH, NWH)], idx_s)
    pltpu.sync_copy(dst_hbm.at[s, pl.ds(c * NWH, NWH)], idx_d)
    pltpu.sync_copy(zeros_hbm, sh.at[pl.ds(s * STR, STR)])
    plsc.subcore_barrier()

    def fire_gathers(k, buf):
        for j in range(CH3):
            pltpu.async_copy(rows_hbm.at[idx_s.at[k * CH3 + j]],
                             buf.at[j], sem0)

    def drain_gathers(k, buf):
        for j in range(CH3):
            pltpu.make_async_copy(rows_hbm.at[idx_s.at[k * CH3 + j]],
                                  buf.at[j], sem0).wait()

    def fire_scatters(k, buf):
        for j in range(CH3):
            pltpu.async_copy(buf.at[j], sh.at[idx_d.at[k * CH3 + j]],
                             sem1, add=True)

    def drain_scatters(k, buf):
        for j in range(CH3):
            pltpu.make_async_copy(buf.at[j], sh.at[idx_d.at[k * CH3 + j]],
                                  sem1).wait()

    ngroups = NWH // CH3
    bufs = (r0, r1)
    fire_gathers(0, r0)
    for k in range(ngroups):
        b = bufs[k % 2]
        drain_gathers(k, b)
        if k + 1 < ngroups:
            fire_gathers(k + 1, bufs[(k + 1) % 2])
        fire_scatters(k, b)
        drain_scatters(k, b)

    plsc.subcore_barrier()
    pltpu.sync_copy(sh.at[pl.ds(s * STR, STR)],
                    out.at[c, pl.ds(s * STR, STR)])


def _tc_norms_body(x_ref, do_ref, di_ref, xs0_ref, xs1_ref, ns_ref, nd_ref):
    ns = lax.rsqrt(jnp.maximum(do_ref[0, :, :1] + do_ref[1, :, :1], 1.0))
    nd = lax.rsqrt(jnp.maximum(di_ref[0, :, :1] + di_ref[1, :, :1], 1.0))
    ns_ref[...] = ns
    nd_ref[...] = nd
    xsc = x_ref[...] * ns
    xs0_ref[...] = xsc[:, :FH]
    xs1_ref[...] = xsc[:, FH:]


def _tc_norms(x_pad, do_p, di_p):
    return pl.pallas_call(
        _tc_norms_body,
        grid=(GRID,),
        in_specs=[
            pl.BlockSpec((BLK, F), lambda i: (i, 0)),
            pl.BlockSpec((NC, BLK, OP), lambda i: (0, i, 0)),
            pl.BlockSpec((NC, BLK, OP), lambda i: (0, i, 0)),
        ],
        out_specs=[
            pl.BlockSpec((BLK, FH), lambda i: (i, 0)),
            pl.BlockSpec((BLK, FH), lambda i: (i, 0)),
            pl.BlockSpec((BLK, 1), lambda i: (i, 0)),
            pl.BlockSpec((BLK, 1), lambda i: (i, 0)),
        ],
        out_shape=[
            jax.ShapeDtypeStruct((NR, FH), jnp.float32),
            jax.ShapeDtypeStruct((NR, FH), jnp.float32),
            jax.ShapeDtypeStruct((NR, 1), jnp.float32),
            jax.ShapeDtypeStruct((NR, 1), jnp.float32),
        ],
    )(x_pad, do_p, di_p)


def _tc_mid_body(agg_ref, nd_ref, w1_ref, b1_ref, w2_ref, ns_ref, out_ref):
    t = jnp.concatenate([agg_ref[0], agg_ref[1]], axis=-1) * nd_ref[...]
    x1 = jnp.dot(t, w1_ref[...], preferred_element_type=jnp.float32)
    x1 = jnp.maximum(x1 + b1_ref[...], 0.0)
    h2 = jnp.dot(x1, w2_ref[...], preferred_element_type=jnp.float32)
    out_ref[...] = h2 * ns_ref[...]


def _tc_mid(agg_h, nd, w1, b1_2d, w2p, ns):
    return pl.pallas_call(
        _tc_mid_body,
        grid=(GRID,),
        in_specs=[
            pl.BlockSpec((NC, BLK, FH), lambda i: (0, i, 0)),
            pl.BlockSpec((BLK, 1), lambda i: (i, 0)),
            pl.BlockSpec((F, F), lambda i: (0, 0)),
            pl.BlockSpec((1, F), lambda i: (0, 0)),
            pl.BlockSpec((F, OP), lambda i: (0, 0)),
            pl.BlockSpec((BLK, 1), lambda i: (i, 0)),
        ],
        out_specs=pl.BlockSpec((BLK, OP), lambda i: (i, 0)),
        out_shape=jax.ShapeDtypeStruct((NR, OP), jnp.float32),
    )(agg_h, nd, w1, b1_2d, w2p, ns)


def _tc_final_body(agg_ref, nd_ref, b2_ref, z_ref, att_ref):
    t = (agg_ref[0] + agg_ref[1]) * nd_ref[...] + b2_ref[...]
    x2 = jnp.maximum(t, 0.0)
    zv = jnp.sum(x2, axis=1, keepdims=True) * (1.0 / 5.0)
    z_ref[...] = zv
    att_ref[...] = jnp.ones((BLK, 1), jnp.float32)


def _tc_final(agg2_p, nd, b2p_2d):
    return pl.pallas_call(
        _tc_final_body,
        grid=(GRID,),
        in_specs=[
            pl.BlockSpec((NC, BLK, OP), lambda i: (0, i, 0)),
            pl.BlockSpec((BLK, 1), lambda i: (i, 0)),
            pl.BlockSpec((1, OP), lambda i: (0, 0)),
        ],
        out_specs=[
            pl.BlockSpec((BLK, 1), lambda i: (i, 0)),
            pl.BlockSpec((BLK, 1), lambda i: (i, 0)),
        ],
        out_shape=[
            jax.ShapeDtypeStruct((NR, 1), jnp.float32),
            jax.ShapeDtypeStruct((NR, 1), jnp.float32),
        ],
    )(agg2_p, nd, b2p_2d)


def _pad_edges(idx):
    per = E // NS
    pad = EPT - per
    r = idx.astype(jnp.int32).reshape(NS, per)
    padv = N + (jnp.arange(pad, dtype=jnp.int32) % GR)
    padv = jnp.broadcast_to(padv, (NS, pad))
    return jnp.concatenate([r, padv], axis=1).reshape(NS, NWIN, WIN)


def kernel(features, edge_index, W1, b1, W2, b2, W_att, b_att):
    n_nodes = features.shape[-1]
    x = jnp.reshape(features, (n_nodes, -1))
    x_pad = jnp.pad(x, ((0, NR - N), (0, 0)))

    src = _pad_edges(edge_index[0])
    dst = _pad_edges(edge_index[1])

    ones_w = jnp.ones((WIN, OP), jnp.float32)
    zeros_s16 = jnp.zeros((STR, OP), jnp.float32)
    zeros_s64 = jnp.zeros((STR, FH), jnp.float32)

    do_p, di_p = _sc_degrees(src, dst, ones_w, zeros_s16)
    xs0, xs1, ns, nd = _tc_norms(x_pad, do_p, di_p)
    agg1_h = _sc_agg128(xs0, xs1, src, dst, zeros_s64)

    b1_2d = jnp.reshape(b1, (1, F))
    w2p = jnp.pad(W2, ((0, 0), (0, OP - W2.shape[1])))
    h2s = _tc_mid(agg1_h, nd, W1, b1_2d, w2p, ns)

    agg2_p = _sc_agg16(h2s, src, dst, zeros_s16)

    b2p_2d = jnp.reshape(jnp.pad(b2, (0, OP - b2.shape[0])), (1, OP))
    z2d, att2d = _tc_final(agg2_p, nd, b2p_2d)

    z = z2d[:N, 0]
    att = att2d[:N]
    return (z, att)


# speedup vs baseline: 16.3437x; 1.1715x over previous
import functools

import jax
import jax.numpy as jnp
from jax import lax
from jax.experimental import pallas as pl
from jax.experimental.pallas import tpu as pltpu
from jax.experimental.pallas import tpu_sc as plsc

N = 10000
E = 320000
F = 128
FH = F // 2
OP = 16
NC, NS = 2, 16
WIN = 128
NWIN = 160
NWH = NWIN // 2
EPT = NWIN * WIN
EP = NS * EPT
GR = 240
NR = N + GR
STR = NR // NS
CH1 = 10
CH3 = 8
BLK = 256
GRID = NR // BLK

_SC_PARAMS = pltpu.CompilerParams(use_tc_tiling_on_sc=False)


def _sc_mesh():
    return plsc.VectorSubcoreMesh(core_axis_name="c", subcore_axis_name="s")


@functools.partial(
    pl.kernel,
    out_type=(
        jax.ShapeDtypeStruct((NC, NR, OP), jnp.float32),
        jax.ShapeDtypeStruct((NC, NR, OP), jnp.float32),
    ),
    mesh=_sc_mesh(),
    compiler_params=_SC_PARAMS,
    scratch_types=(
        pltpu.VMEM((NWH, WIN), jnp.int32),
        pltpu.VMEM((NWH, WIN), jnp.int32),
        pltpu.VMEM((WIN, OP), jnp.float32),
        pltpu.VMEM_SHARED((NR, OP), jnp.float32),
        pltpu.VMEM_SHARED((NR, OP), jnp.float32),
        pltpu.SemaphoreType.DMA,
    ),
)
def _sc_degrees(src_hbm, dst_hbm, ones_hbm, zeros_hbm, do_out, di_out,
                idx_s, idx_d, ones_v, sh_do, sh_di, sem):
    c = lax.axis_index("c")
    s = lax.axis_index("s")
    pltpu.sync_copy(src_hbm.at[s, pl.ds(c * NWH, NWH)], idx_s)
    pltpu.sync_copy(dst_hbm.at[s, pl.ds(c * NWH, NWH)], idx_d)
    pltpu.sync_copy(ones_hbm, ones_v)
    pltpu.sync_copy(zeros_hbm, sh_do.at[pl.ds(s * STR, STR)])
    pltpu.sync_copy(zeros_hbm, sh_di.at[pl.ds(s * STR, STR)])
    plsc.subcore_barrier()

    @pl.loop(0, NWH // CH1)
    def _(k):
        for j in range(CH1):
            w = k * CH1 + j
            pltpu.async_copy(ones_v, sh_do.at[idx_s.at[w]], sem, add=True)
            pltpu.async_copy(ones_v, sh_di.at[idx_d.at[w]], sem, add=True)
        for j in range(CH1):
            w = k * CH1 + j
            pltpu.make_async_copy(ones_v, sh_do.at[idx_s.at[w]], sem).wait()
            pltpu.make_async_copy(ones_v, sh_di.at[idx_d.at[w]], sem).wait()

    plsc.subcore_barrier()
    pltpu.sync_copy(sh_do.at[pl.ds(s * STR, STR)],
                    do_out.at[c, pl.ds(s * STR, STR)])
    pltpu.sync_copy(sh_di.at[pl.ds(s * STR, STR)],
                    di_out.at[c, pl.ds(s * STR, STR)])


@functools.partial(
    pl.kernel,
    out_type=jax.ShapeDtypeStruct((NC, NR, FH), jnp.float32),
    mesh=_sc_mesh(),
    compiler_params=_SC_PARAMS,
    scratch_types=(
        pltpu.VMEM((NWIN, WIN), jnp.int32),
        pltpu.VMEM((NWIN, WIN), jnp.int32),
        pltpu.VMEM((4, WIN, FH), jnp.float32),
        pltpu.VMEM_SHARED((NR, FH), jnp.float32),
        pltpu.SemaphoreType.DMA,
        pltpu.SemaphoreType.DMA,
    ),
)
def _sc_agg128(xs0_hbm, xs1_hbm, src_hbm, dst_hbm, zeros_hbm, out,
               idx_s, idx_d, rb, sh, gsem, ssem):
    c = lax.axis_index("c")
    s = lax.axis_index("s")
    pltpu.sync_copy(src_hbm.at[s], idx_s)
    pltpu.sync_copy(dst_hbm.at[s], idx_d)
    pltpu.sync_copy(zeros_hbm, sh.at[pl.ds(s * STR, STR)])
    plsc.subcore_barrier()

    def run_half(xs_ref):
        for j in range(4):
            pltpu.async_copy(xs_ref.at[idx_s.at[j]], rb.at[j], gsem)

        @pl.loop(0, NWIN // 4)
        def _(t):
            w0 = 4 * t
            for j in range(4):
                w = w0 + j
                pltpu.make_async_copy(xs_ref.at[idx_s.at[w]],
                                      rb.at[j], gsem).wait()
                pltpu.async_copy(rb.at[j], sh.at[idx_d.at[w]], ssem,
                                 add=True)
            for j in range(4):
                w = w0 + j

                @pl.when(w + 4 < NWIN)
                def _():
                    pltpu.make_async_copy(rb.at[j], sh.at[idx_d.at[w]],
                                          ssem).wait()
                    pltpu.async_copy(xs_ref.at[idx_s.at[w + 4]],
                                     rb.at[j], gsem)

        for j in range(4):
            w = NWIN - 4 + j
            pltpu.make_async_copy(rb.at[j], sh.at[idx_d.at[w]], ssem).wait()

    @pl.when(c == 0)
    def _():
        run_half(xs0_hbm)

    @pl.when(c == 1)
    def _():
        run_half(xs1_hbm)

    plsc.subcore_barrier()
    pltpu.sync_copy(sh.at[pl.ds(s * STR, STR)],
                    out.at[c, pl.ds(s * STR, STR)])


@functools.partial(
    pl.kernel,
    out_type=jax.ShapeDtypeStruct((NC, NR, OP), jnp.float32),
    mesh=_sc_mesh(),
    compiler_params=_SC_PARAMS,
    scratch_types=(
        pltpu.VMEM((NWH, WIN), jnp.int32),
        pltpu.VMEM((NWH, WIN), jnp.int32),
        pltpu.VMEM((CH3, WIN, OP), jnp.float32),
        pltpu.VMEM((CH3, WIN, OP), jnp.float32),
        pltpu.VMEM_SHARED((NR, OP), jnp.float32),
        pltpu.SemaphoreType.DMA,
        pltpu.SemaphoreType.DMA,
    ),
)
def _sc_agg16(rows_hbm, src_hbm, dst_hbm, zeros_hbm, out,
              idx_s, idx_d, r0, r1, sh, sem0, sem1):
    c = lax.axis_index("c")
    s = lax.axis_index("s")
    pltpu.sync_copy(src_hbm.at[s, pl.ds(c * NWH, NWH)], idx_s)
    pltpu.sync_copy(dst_hbm.at[s, pl.ds(c * NWH, NWH)], idx_d)
    pltpu.sync_copy(zeros_hbm, sh.at[pl.ds(s * STR, STR)])
    plsc.subcore_barrier()

    def fire_gathers(k, buf):
        for j in range(CH3):
            pltpu.async_copy(rows_hbm.at[idx_s.at[k * CH3 + j]],
                             buf.at[j], sem0)

    def drain_gathers(k, buf):
        for j in range(CH3):
            pltpu.make_async_copy(rows_hbm.at[idx_s.at[k * CH3 + j]],
                                  buf.at[j], sem0).wait()

    def fire_scatters(k, buf):
        for j in range(CH3):
            pltpu.async_copy(buf.at[j], sh.at[idx_d.at[k * CH3 + j]],
                             sem1, add=True)

    def drain_scatters(k, buf):
        for j in range(CH3):
            pltpu.make_async_copy(buf.at[j], sh.at[idx_d.at[k * CH3 + j]],
                                  sem1).wait()

    ngroups = NWH // CH3
    bufs = (r0, r1)
    fire_gathers(0, r0)
    for k in range(ngroups):
        b = bufs[k % 2]
        drain_gathers(k, b)
        if k + 1 < ngroups:
            fire_gathers(k + 1, bufs[(k + 1) % 2])
        fire_scatters(k, b)
        drain_scatters(k, b)

    plsc.subcore_barrier()
    pltpu.sync_copy(sh.at[pl.ds(s * STR, STR)],
                    out.at[c, pl.ds(s * STR, STR)])


def _tc_norms_body(x_ref, do_ref, di_ref, xs0_ref, xs1_ref, ns_ref, nd_ref):
    ns = lax.rsqrt(jnp.maximum(do_ref[0, :, :1] + do_ref[1, :, :1], 1.0))
    nd = lax.rsqrt(jnp.maximum(di_ref[0, :, :1] + di_ref[1, :, :1], 1.0))
    ns_ref[...] = ns
    nd_ref[...] = nd
    xsc = x_ref[...] * ns
    xs0_ref[...] = xsc[:, :FH]
    xs1_ref[...] = xsc[:, FH:]


def _tc_norms(x_pad, do_p, di_p):
    return pl.pallas_call(
        _tc_norms_body,
        grid=(GRID,),
        in_specs=[
            pl.BlockSpec((BLK, F), lambda i: (i, 0)),
            pl.BlockSpec((NC, BLK, OP), lambda i: (0, i, 0)),
            pl.BlockSpec((NC, BLK, OP), lambda i: (0, i, 0)),
        ],
        out_specs=[
            pl.BlockSpec((BLK, FH), lambda i: (i, 0)),
            pl.BlockSpec((BLK, FH), lambda i: (i, 0)),
            pl.BlockSpec((BLK, 1), lambda i: (i, 0)),
            pl.BlockSpec((BLK, 1), lambda i: (i, 0)),
        ],
        out_shape=[
            jax.ShapeDtypeStruct((NR, FH), jnp.float32),
            jax.ShapeDtypeStruct((NR, FH), jnp.float32),
            jax.ShapeDtypeStruct((NR, 1), jnp.float32),
            jax.ShapeDtypeStruct((NR, 1), jnp.float32),
        ],
    )(x_pad, do_p, di_p)


def _tc_mid_body(agg_ref, nd_ref, w1_ref, b1_ref, w2_ref, ns_ref, out_ref):
    t = jnp.concatenate([agg_ref[0], agg_ref[1]], axis=-1) * nd_ref[...]
    x1 = jnp.dot(t, w1_ref[...], preferred_element_type=jnp.float32)
    x1 = jnp.maximum(x1 + b1_ref[...], 0.0)
    h2 = jnp.dot(x1, w2_ref[...], preferred_element_type=jnp.float32)
    out_ref[...] = h2 * ns_ref[...]


def _tc_mid(agg_h, nd, w1, b1_2d, w2p, ns):
    return pl.pallas_call(
        _tc_mid_body,
        grid=(GRID,),
        in_specs=[
            pl.BlockSpec((NC, BLK, FH), lambda i: (0, i, 0)),
            pl.BlockSpec((BLK, 1), lambda i: (i, 0)),
            pl.BlockSpec((F, F), lambda i: (0, 0)),
            pl.BlockSpec((1, F), lambda i: (0, 0)),
            pl.BlockSpec((F, OP), lambda i: (0, 0)),
            pl.BlockSpec((BLK, 1), lambda i: (i, 0)),
        ],
        out_specs=pl.BlockSpec((BLK, OP), lambda i: (i, 0)),
        out_shape=jax.ShapeDtypeStruct((NR, OP), jnp.float32),
    )(agg_h, nd, w1, b1_2d, w2p, ns)


def _tc_final_body(agg_ref, nd_ref, b2_ref, z_ref, att_ref):
    t = (agg_ref[0] + agg_ref[1]) * nd_ref[...] + b2_ref[...]
    x2 = jnp.maximum(t, 0.0)
    zv = jnp.sum(x2, axis=1, keepdims=True) * (1.0 / 5.0)
    z_ref[...] = zv
    att_ref[...] = jnp.ones((BLK, 1), jnp.float32)


def _tc_final(agg2_p, nd, b2p_2d):
    return pl.pallas_call(
        _tc_final_body,
        grid=(GRID,),
        in_specs=[
            pl.BlockSpec((NC, BLK, OP), lambda i: (0, i, 0)),
            pl.BlockSpec((BLK, 1), lambda i: (i, 0)),
            pl.BlockSpec((1, OP), lambda i: (0, 0)),
        ],
        out_specs=[
            pl.BlockSpec((BLK, 1), lambda i: (i, 0)),
            pl.BlockSpec((BLK, 1), lambda i: (i, 0)),
        ],
        out_shape=[
            jax.ShapeDtypeStruct((NR, 1), jnp.float32),
            jax.ShapeDtypeStruct((NR, 1), jnp.float32),
        ],
    )(agg2_p, nd, b2p_2d)


def _pad_edges(idx):
    per = E // NS
    pad = EPT - per
    r = idx.astype(jnp.int32).reshape(NS, per)
    padv = N + (jnp.arange(pad, dtype=jnp.int32) % GR)
    padv = jnp.broadcast_to(padv, (NS, pad))
    return jnp.concatenate([r, padv], axis=1).reshape(NS, NWIN, WIN)


def kernel(features, edge_index, W1, b1, W2, b2, W_att, b_att):
    n_nodes = features.shape[-1]
    x = jnp.reshape(features, (n_nodes, -1))
    x_pad = jnp.pad(x, ((0, NR - N), (0, 0)))

    src = _pad_edges(edge_index[0])
    dst = _pad_edges(edge_index[1])

    ones_w = jnp.ones((WIN, OP), jnp.float32)
    zeros_s16 = jnp.zeros((STR, OP), jnp.float32)
    zeros_s64 = jnp.zeros((STR, FH), jnp.float32)

    do_p, di_p = _sc_degrees(src, dst, ones_w, zeros_s16)
    xs0, xs1, ns, nd = _tc_norms(x_pad, do_p, di_p)
    agg1_h = _sc_agg128(xs0, xs1, src, dst, zeros_s64)

    b1_2d = jnp.reshape(b1, (1, F))
    w2p = jnp.pad(W2, ((0, 0), (0, OP - W2.shape[1])))
    h2s = _tc_mid(agg1_h, nd, W1, b1_2d, w2p, ns)

    agg2_p = _sc_agg16(h2s, src, dst, zeros_s16)

    b2p_2d = jnp.reshape(jnp.pad(b2, (0, OP - b2.shape[0])), (1, OP))
    z2d, att2d = _tc_final(agg2_p, nd, b2p_2d)

    z = z2d[:N, 0]
    att = att2d[:N]
    return (z, att)


# speedup vs baseline: 16.5450x; 1.0123x over previous
import functools

import jax
import jax.numpy as jnp
from jax import lax
from jax.experimental import pallas as pl
from jax.experimental.pallas import tpu as pltpu
from jax.experimental.pallas import tpu_sc as plsc

N = 10000
E = 320000
F = 128
FH = F // 2
OP = 16
NC, NS = 2, 16
WIN = 128
NWIN = 160
NWH = NWIN // 2
EPT = NWIN * WIN
EP = NS * EPT
GR = 240
NR = N + GR
STR = NR // NS
CH1 = 10
CH3 = 8
BLK = 256
GRID = NR // BLK

_SC_PARAMS = pltpu.CompilerParams(use_tc_tiling_on_sc=False)
_SC_PARAMS_NLP = pltpu.CompilerParams(use_tc_tiling_on_sc=False,
                                      needs_layout_passes=False)


def _sc_mesh():
    return plsc.VectorSubcoreMesh(core_axis_name="c", subcore_axis_name="s")


def _rsqrt16(d):
    h = d * 0.5
    i = plsc.bitcast(d, jnp.int32)
    i = jnp.int32(0x5F3759DF) - (i >> 1)
    y = plsc.bitcast(i, jnp.float32)
    for _ in range(3):
        y = y * (1.5 - h * y * y)
    return y


@functools.partial(
    pl.kernel,
    out_type=(
        jax.ShapeDtypeStruct((NR, FH), jnp.float32),
        jax.ShapeDtypeStruct((NR, FH), jnp.float32),
        jax.ShapeDtypeStruct((NR, OP), jnp.float32),
        jax.ShapeDtypeStruct((NR, OP), jnp.float32),
    ),
    mesh=_sc_mesh(),
    compiler_params=_SC_PARAMS_NLP,
    scratch_types=(
        pltpu.VMEM((NWIN, WIN), jnp.int32),
        pltpu.VMEM((WIN, OP), jnp.float32),
        pltpu.VMEM((STR, OP), jnp.float32),
        pltpu.VMEM((WIN, FH), jnp.float32),
        pltpu.VMEM((WIN, FH), jnp.float32),
        pltpu.VMEM_SHARED((NR, OP), jnp.float32),
        pltpu.SemaphoreType.DMA,
    ),
)
def _sc_prep(x0_hbm, x1_hbm, src_hbm, dst_hbm, ones_hbm, zeros_hbm,
             xs0_out, xs1_out, ns_out, nd_out,
             idx, ones_v, nsv, xb0, xb1, sh_deg, sem):
    c = lax.axis_index("c")
    s = lax.axis_index("s")

    @pl.when(c == 0)
    def _():
        pltpu.sync_copy(src_hbm.at[s], idx)

    @pl.when(c == 1)
    def _():
        pltpu.sync_copy(dst_hbm.at[s], idx)

    pltpu.sync_copy(ones_hbm, ones_v)
    pltpu.sync_copy(zeros_hbm, sh_deg.at[pl.ds(s * STR, STR)])
    plsc.subcore_barrier()

    @pl.loop(0, NWIN // CH1)
    def _(k):
        for j in range(CH1):
            w = k * CH1 + j
            pltpu.async_copy(ones_v, sh_deg.at[idx.at[w]], sem, add=True)
        for j in range(CH1):
            w = k * CH1 + j
            pltpu.make_async_copy(ones_v, sh_deg.at[idx.at[w]], sem).wait()

    plsc.subcore_barrier()
    pltpu.sync_copy(sh_deg.at[pl.ds(s * STR, STR)], nsv)

    @pl.loop(0, STR)
    def _(r):
        nsv[r] = _rsqrt16(jnp.maximum(nsv[r], 1.0))

    @pl.when(c == 0)
    def _():
        pltpu.sync_copy(nsv, ns_out.at[pl.ds(s * STR, STR)])
        for t in range(STR // WIN):
            base = s * STR + t * WIN
            pltpu.sync_copy(x0_hbm.at[pl.ds(base, WIN)], xb0)
            pltpu.sync_copy(x1_hbm.at[pl.ds(base, WIN)], xb1)

            @pl.loop(0, WIN)
            def _(r):
                nv = nsv[t * WIN + r]
                for q in range(FH // 16):
                    xb0[r, pl.ds(q * 16, 16)] = xb0[r, pl.ds(q * 16, 16)] * nv
                    xb1[r, pl.ds(q * 16, 16)] = xb1[r, pl.ds(q * 16, 16)] * nv

            pltpu.sync_copy(xb0, xs0_out.at[pl.ds(base, WIN)])
            pltpu.sync_copy(xb1, xs1_out.at[pl.ds(base, WIN)])

    @pl.when(c == 1)
    def _():
        pltpu.sync_copy(nsv, nd_out.at[pl.ds(s * STR, STR)])


@functools.partial(
    pl.kernel,
    out_type=jax.ShapeDtypeStruct((NC, NR, FH), jnp.float32),
    mesh=_sc_mesh(),
    compiler_params=_SC_PARAMS,
    scratch_types=(
        pltpu.VMEM((NWIN, WIN), jnp.int32),
        pltpu.VMEM((NWIN, WIN), jnp.int32),
        pltpu.VMEM((4, WIN, FH), jnp.float32),
        pltpu.VMEM_SHARED((NR, FH), jnp.float32),
        pltpu.SemaphoreType.DMA,
        pltpu.SemaphoreType.DMA,
    ),
)
def _sc_agg128(xs0_hbm, xs1_hbm, src_hbm, dst_hbm, zeros_hbm, out,
               idx_s, idx_d, rb, sh, gsem, ssem):
    c = lax.axis_index("c")
    s = lax.axis_index("s")
    pltpu.sync_copy(src_hbm.at[s], idx_s)
    pltpu.sync_copy(dst_hbm.at[s], idx_d)
    pltpu.sync_copy(zeros_hbm, sh.at[pl.ds(s * STR, STR)])
    plsc.subcore_barrier()

    def run_half(xs_ref):
        for j in range(4):
            pltpu.async_copy(xs_ref.at[idx_s.at[j]], rb.at[j], gsem)

        @pl.loop(0, NWIN // 4)
        def _(t):
            w0 = 4 * t
            for j in range(4):
                w = w0 + j
                pltpu.make_async_copy(xs_ref.at[idx_s.at[w]],
                                      rb.at[j], gsem).wait()
                pltpu.async_copy(rb.at[j], sh.at[idx_d.at[w]], ssem,
                                 add=True)
            for j in range(4):
                w = w0 + j

                @pl.when(w + 4 < NWIN)
                def _():
                    pltpu.make_async_copy(rb.at[j], sh.at[idx_d.at[w]],
                                          ssem).wait()
                    pltpu.async_copy(xs_ref.at[idx_s.at[w + 4]],
                                     rb.at[j], gsem)

        for j in range(4):
            w = NWIN - 4 + j
            pltpu.make_async_copy(rb.at[j], sh.at[idx_d.at[w]], ssem).wait()

    @pl.when(c == 0)
    def _():
        run_half(xs0_hbm)

    @pl.when(c == 1)
    def _():
        run_half(xs1_hbm)

    plsc.subcore_barrier()
    pltpu.sync_copy(sh.at[pl.ds(s * STR, STR)],
                    out.at[c, pl.ds(s * STR, STR)])


@functools.partial(
    pl.kernel,
    out_type=jax.ShapeDtypeStruct((NC, NR, OP), jnp.float32),
    mesh=_sc_mesh(),
    compiler_params=_SC_PARAMS,
    scratch_types=(
        pltpu.VMEM((NWH, WIN), jnp.int32),
        pltpu.VMEM((NWH, WIN), jnp.int32),
        pltpu.VMEM((CH3, WIN, OP), jnp.float32),
        pltpu.VMEM((CH3, WIN, OP), jnp.float32),
        pltpu.VMEM_SHARED((NR, OP), jnp.float32),
        pltpu.SemaphoreType.DMA,
        pltpu.SemaphoreType.DMA,
    ),
)
def _sc_agg16(rows_hbm, src_hbm, dst_hbm, zeros_hbm, out,
              idx_s, idx_d, r0, r1, sh, sem0, sem1):
    c = lax.axis_index("c")
    s = lax.axis_index("s")
    pltpu.sync_copy(src_hbm.at[s, pl.ds(c * NWH, NWH)], idx_s)
    pltpu.sync_copy(dst_hbm.at[s, pl.ds(c * NWH, NWH)], idx_d)
    pltpu.sync_copy(zeros_hbm, sh.at[pl.ds(s * STR, STR)])
    plsc.subcore_barrier()

    def fire_gathers(k, buf):
        for j in range(CH3):
            pltpu.async_copy(rows_hbm.at[idx_s.at[k * CH3 + j]],
                             buf.at[j], sem0)

    def drain_gathers(k, buf):
        for j in range(CH3):
            pltpu.make_async_copy(rows_hbm.at[idx_s.at[k * CH3 + j]],
                                  buf.at[j], sem0).wait()

    def fire_scatters(k, buf):
        for j in range(CH3):
            pltpu.async_copy(buf.at[j], sh.at[idx_d.at[k * CH3 + j]],
                             sem1, add=True)

    def drain_scatters(k, buf):
        for j in range(CH3):
            pltpu.make_async_copy(buf.at[j], sh.at[idx_d.at[k * CH3 + j]],
                                  sem1).wait()

    ngroups = NWH // CH3
    bufs = (r0, r1)
    fire_gathers(0, r0)
    for k in range(ngroups):
        b = bufs[k % 2]
        drain_gathers(k, b)
        if k + 1 < ngroups:
            fire_gathers(k + 1, bufs[(k + 1) % 2])
        fire_scatters(k, b)
        drain_scatters(k, b)

    plsc.subcore_barrier()
    pltpu.sync_copy(sh.at[pl.ds(s * STR, STR)],
                    out.at[c, pl.ds(s * STR, STR)])


def _tc_mid_body(agg_ref, nd_ref, w1_ref, b1_ref, w2_ref, ns_ref, out_ref):
    t = jnp.concatenate([agg_ref[0], agg_ref[1]], axis=-1) * nd_ref[:, :1]
    x1 = jnp.dot(t, w1_ref[...], preferred_element_type=jnp.float32)
    x1 = jnp.maximum(x1 + b1_ref[...], 0.0)
    h2 = jnp.dot(x1, w2_ref[...], preferred_element_type=jnp.float32)
    out_ref[...] = h2 * ns_ref[:, :1]


def _tc_mid(agg_h, nd, w1, b1_2d, w2p, ns):
    return pl.pallas_call(
        _tc_mid_body,
        grid=(GRID,),
        in_specs=[
            pl.BlockSpec((NC, BLK, FH), lambda i: (0, i, 0)),
            pl.BlockSpec((BLK, OP), lambda i: (i, 0)),
            pl.BlockSpec((F, F), lambda i: (0, 0)),
            pl.BlockSpec((1, F), lambda i: (0, 0)),
            pl.BlockSpec((F, OP), lambda i: (0, 0)),
            pl.BlockSpec((BLK, OP), lambda i: (i, 0)),
        ],
        out_specs=pl.BlockSpec((BLK, OP), lambda i: (i, 0)),
        out_shape=jax.ShapeDtypeStruct((NR, OP), jnp.float32),
    )(agg_h, nd, w1, b1_2d, w2p, ns)


def _tc_final_body(agg_ref, nd_ref, b2_ref, z_ref, att_ref):
    t = (agg_ref[0] + agg_ref[1]) * nd_ref[:, :1] + b2_ref[...]
    x2 = jnp.maximum(t, 0.0)
    zv = jnp.sum(x2, axis=1, keepdims=True) * (1.0 / 5.0)
    z_ref[...] = zv
    att_ref[...] = jnp.ones((BLK, 1), jnp.float32)


def _tc_final(agg2_p, nd, b2p_2d):
    return pl.pallas_call(
        _tc_final_body,
        grid=(GRID,),
        in_specs=[
            pl.BlockSpec((NC, BLK, OP), lambda i: (0, i, 0)),
            pl.BlockSpec((BLK, OP), lambda i: (i, 0)),
            pl.BlockSpec((1, OP), lambda i: (0, 0)),
        ],
        out_specs=[
            pl.BlockSpec((BLK, 1), lambda i: (i, 0)),
            pl.BlockSpec((BLK, 1), lambda i: (i, 0)),
        ],
        out_shape=[
            jax.ShapeDtypeStruct((NR, 1), jnp.float32),
            jax.ShapeDtypeStruct((NR, 1), jnp.float32),
        ],
    )(agg2_p, nd, b2p_2d)


def _pad_edges(idx):
    per = E // NS
    pad = EPT - per
    r = idx.astype(jnp.int32).reshape(NS, per)
    padv = N + (jnp.arange(pad, dtype=jnp.int32) % GR)
    padv = jnp.broadcast_to(padv, (NS, pad))
    return jnp.concatenate([r, padv], axis=1).reshape(NS, NWIN, WIN)


def kernel(features, edge_index, W1, b1, W2, b2, W_att, b_att):
    n_nodes = features.shape[-1]
    x = jnp.reshape(features, (n_nodes, -1))
    x_pad = jnp.pad(x, ((0, NR - N), (0, 0)))

    src = _pad_edges(edge_index[0])
    dst = _pad_edges(edge_index[1])

    ones_w = jnp.ones((WIN, OP), jnp.float32)
    zeros_s16 = jnp.zeros((STR, OP), jnp.float32)
    zeros_s64 = jnp.zeros((STR, FH), jnp.float32)

    x0 = x_pad[:, :FH]
    x1 = x_pad[:, FH:]
    xs0, xs1, ns, nd = _sc_prep(x0, x1, src, dst, ones_w, zeros_s16)
    agg1_h = _sc_agg128(xs0, xs1, src, dst, zeros_s64)

    b1_2d = jnp.reshape(b1, (1, F))
    w2p = jnp.pad(W2, ((0, 0), (0, OP - W2.shape[1])))
    h2s = _tc_mid(agg1_h, nd, W1, b1_2d, w2p, ns)

    agg2_p = _sc_agg16(h2s, src, dst, zeros_s16)

    b2p_2d = jnp.reshape(jnp.pad(b2, (0, OP - b2.shape[0])), (1, OP))
    z2d, att2d = _tc_final(agg2_p, nd, b2p_2d)

    z = z2d[:N, 0]
    att = att2d[:N]
    return (z, att)


# speedup vs baseline: 18.1672x; 1.0980x over previous
import functools

import jax
import jax.numpy as jnp
from jax import lax
from jax.experimental import pallas as pl
from jax.experimental.pallas import tpu as pltpu
from jax.experimental.pallas import tpu_sc as plsc

N = 10000
E = 320000
F = 128
FH = F // 2
OP = 16
NC, NS = 2, 16
WIN = 128
NWIN = 160
NWH = NWIN // 2
EPT = NWIN * WIN
EP = NS * EPT
GR = 240
NR = N + GR
STR = NR // NS
CH1 = 10
CH3 = 8
BLK = 256
GRID = NR // BLK

_PACK_PERM = tuple(
    h * 64 + b * 32 + (p // 2) + 16 * (p % 2)
    for h in range(2) for b in range(2) for p in range(32)
)

_SC_PARAMS = pltpu.CompilerParams(use_tc_tiling_on_sc=False)
_SC_PARAMS_NLP = pltpu.CompilerParams(use_tc_tiling_on_sc=False,
                                      needs_layout_passes=False)


def _sc_mesh():
    return plsc.VectorSubcoreMesh(core_axis_name="c", subcore_axis_name="s")


def _rsqrt16(d):
    h = d * 0.5
    i = plsc.bitcast(d, jnp.int32)
    i = jnp.int32(0x5F3759DF) - (i >> 1)
    y = plsc.bitcast(i, jnp.float32)
    for _ in range(3):
        y = y * (1.5 - h * y * y)
    return y


@functools.partial(
    pl.kernel,
    out_type=(
        jax.ShapeDtypeStruct((NR, FH), jnp.bfloat16),
        jax.ShapeDtypeStruct((NR, FH), jnp.bfloat16),
        jax.ShapeDtypeStruct((NR, OP), jnp.float32),
        jax.ShapeDtypeStruct((NR, OP), jnp.float32),
    ),
    mesh=_sc_mesh(),
    compiler_params=_SC_PARAMS_NLP,
    scratch_types=(
        pltpu.VMEM((NWIN, WIN), jnp.int32),
        pltpu.VMEM((WIN, OP), jnp.float32),
        pltpu.VMEM((STR, OP), jnp.float32),
        pltpu.VMEM((WIN, FH), jnp.float32),
        pltpu.VMEM((WIN, FH), jnp.float32),
        pltpu.VMEM((WIN, FH), jnp.bfloat16),
        pltpu.VMEM((WIN, FH), jnp.bfloat16),
        pltpu.VMEM_SHARED((NR, OP), jnp.float32),
        pltpu.SemaphoreType.DMA,
    ),
)
def _sc_prep(x0_hbm, x1_hbm, src_hbm, dst_hbm, ones_hbm, zeros_hbm,
             xs0_out, xs1_out, ns_out, nd_out,
             idx, ones_v, nsv, xb0, xb1, xbb0, xbb1, sh_deg, sem):
    c = lax.axis_index("c")
    s = lax.axis_index("s")

    @pl.when(c == 0)
    def _():
        pltpu.sync_copy(src_hbm.at[s], idx)

    @pl.when(c == 1)
    def _():
        pltpu.sync_copy(dst_hbm.at[s], idx)

    pltpu.sync_copy(ones_hbm, ones_v)
    pltpu.sync_copy(zeros_hbm, sh_deg.at[pl.ds(s * STR, STR)])
    plsc.subcore_barrier()

    @pl.loop(0, NWIN // CH1)
    def _(k):
        for j in range(CH1):
            w = k * CH1 + j
            pltpu.async_copy(ones_v, sh_deg.at[idx.at[w]], sem, add=True)
        for j in range(CH1):
            w = k * CH1 + j
            pltpu.make_async_copy(ones_v, sh_deg.at[idx.at[w]], sem).wait()

    plsc.subcore_barrier()
    pltpu.sync_copy(sh_deg.at[pl.ds(s * STR, STR)], nsv)

    @pl.loop(0, STR)
    def _(r):
        nsv[r] = _rsqrt16(jnp.maximum(nsv[r], 1.0))

    @pl.when(c == 0)
    def _():
        pltpu.sync_copy(nsv, ns_out.at[pl.ds(s * STR, STR)])
        for t in range(STR // WIN):
            base = s * STR + t * WIN
            pltpu.sync_copy(x0_hbm.at[pl.ds(base, WIN)], xb0)
            pltpu.sync_copy(x1_hbm.at[pl.ds(base, WIN)], xb1)

            @pl.loop(0, WIN)
            def _(r):
                nv = nsv[t * WIN + r]
                for src_b, dst_b in ((xb0, xbb0), (xb1, xbb1)):
                    for q in range(FH // 32):
                        a = src_b[r, pl.ds(q * 32, 16)] * nv
                        b = src_b[r, pl.ds(q * 32 + 16, 16)] * nv
                        dst_b[r, pl.ds(q * 32, 32)] = plsc.pack(
                            a, b, format=plsc.PackFormat.INTERLEAVED)

            pltpu.sync_copy(xbb0, xs0_out.at[pl.ds(base, WIN)])
            pltpu.sync_copy(xbb1, xs1_out.at[pl.ds(base, WIN)])

    @pl.when(c == 1)
    def _():
        pltpu.sync_copy(nsv, nd_out.at[pl.ds(s * STR, STR)])


@functools.partial(
    pl.kernel,
    out_type=jax.ShapeDtypeStruct((NC, NR, FH), jnp.bfloat16),
    mesh=_sc_mesh(),
    compiler_params=_SC_PARAMS,
    scratch_types=(
        pltpu.VMEM((NWIN, WIN), jnp.int32),
        pltpu.VMEM((NWIN, WIN), jnp.int32),
        pltpu.VMEM((4, WIN, FH), jnp.bfloat16),
        pltpu.VMEM_SHARED((NR, FH), jnp.bfloat16),
        pltpu.SemaphoreType.DMA,
        pltpu.SemaphoreType.DMA,
    ),
)
def _sc_agg128(xs0_hbm, xs1_hbm, src_hbm, dst_hbm, zeros_hbm, out,
               idx_s, idx_d, rb, sh, gsem, ssem):
    c = lax.axis_index("c")
    s = lax.axis_index("s")
    pltpu.sync_copy(src_hbm.at[s], idx_s)
    pltpu.sync_copy(dst_hbm.at[s], idx_d)
    pltpu.sync_copy(zeros_hbm, sh.at[pl.ds(s * STR, STR)])
    plsc.subcore_barrier()

    def run_half(xs_ref):
        for j in range(4):
            pltpu.async_copy(xs_ref.at[idx_s.at[j]], rb.at[j], gsem)

        @pl.loop(0, NWIN // 4)
        def _(t):
            w0 = 4 * t
            for j in range(4):
                w = w0 + j
                pltpu.make_async_copy(xs_ref.at[idx_s.at[w]],
                                      rb.at[j], gsem).wait()
                pltpu.async_copy(rb.at[j], sh.at[idx_d.at[w]], ssem,
                                 add=True)
            for j in range(4):
                w = w0 + j

                @pl.when(w + 4 < NWIN)
                def _():
                    pltpu.make_async_copy(rb.at[j], sh.at[idx_d.at[w]],
                                          ssem).wait()
                    pltpu.async_copy(xs_ref.at[idx_s.at[w + 4]],
                                     rb.at[j], gsem)

        for j in range(4):
            w = NWIN - 4 + j
            pltpu.make_async_copy(rb.at[j], sh.at[idx_d.at[w]], ssem).wait()

    @pl.when(c == 0)
    def _():
        run_half(xs0_hbm)

    @pl.when(c == 1)
    def _():
        run_half(xs1_hbm)

    plsc.subcore_barrier()
    pltpu.sync_copy(sh.at[pl.ds(s * STR, STR)],
                    out.at[c, pl.ds(s * STR, STR)])


@functools.partial(
    pl.kernel,
    out_type=jax.ShapeDtypeStruct((NC, NR, OP), jnp.float32),
    mesh=_sc_mesh(),
    compiler_params=_SC_PARAMS,
    scratch_types=(
        pltpu.VMEM((NWH, WIN), jnp.int32),
        pltpu.VMEM((NWH, WIN), jnp.int32),
        pltpu.VMEM((CH3, WIN, OP), jnp.float32),
        pltpu.VMEM((CH3, WIN, OP), jnp.float32),
        pltpu.VMEM_SHARED((NR, OP), jnp.float32),
        pltpu.SemaphoreType.DMA,
        pltpu.SemaphoreType.DMA,
    ),
)
def _sc_agg16(rows_hbm, src_hbm, dst_hbm, zeros_hbm, out,
              idx_s, idx_d, r0, r1, sh, sem0, sem1):
    c = lax.axis_index("c")
    s = lax.axis_index("s")
    pltpu.sync_copy(src_hbm.at[s, pl.ds(c * NWH, NWH)], idx_s)
    pltpu.sync_copy(dst_hbm.at[s, pl.ds(c * NWH, NWH)], idx_d)
    pltpu.sync_copy(zeros_hbm, sh.at[pl.ds(s * STR, STR)])
    plsc.subcore_barrier()

    def fire_gathers(k, buf):
        for j in range(CH3):
            pltpu.async_copy(rows_hbm.at[idx_s.at[k * CH3 + j]],
                             buf.at[j], sem0)

    def drain_gathers(k, buf):
        for j in range(CH3):
            pltpu.make_async_copy(rows_hbm.at[idx_s.at[k * CH3 + j]],
                                  buf.at[j], sem0).wait()

    def fire_scatters(k, buf):
        for j in range(CH3):
            pltpu.async_copy(buf.at[j], sh.at[idx_d.at[k * CH3 + j]],
                             sem1, add=True)

    def drain_scatters(k, buf):
        for j in range(CH3):
            pltpu.make_async_copy(buf.at[j], sh.at[idx_d.at[k * CH3 + j]],
                                  sem1).wait()

    ngroups = NWH // CH3
    bufs = (r0, r1)
    fire_gathers(0, r0)
    for k in range(ngroups):
        b = bufs[k % 2]
        drain_gathers(k, b)
        if k + 1 < ngroups:
            fire_gathers(k + 1, bufs[(k + 1) % 2])
        fire_scatters(k, b)
        drain_scatters(k, b)

    plsc.subcore_barrier()
    pltpu.sync_copy(sh.at[pl.ds(s * STR, STR)],
                    out.at[c, pl.ds(s * STR, STR)])


def _tc_mid_body(agg_ref, nd_ref, w1_ref, b1_ref, w2_ref, ns_ref, out_ref):
    t = jnp.concatenate([agg_ref[0], agg_ref[1]], axis=-1)
    t = t.astype(jnp.float32) * nd_ref[:, :1]
    x1 = jnp.dot(t, w1_ref[...], preferred_element_type=jnp.float32)
    x1 = jnp.maximum(x1 + b1_ref[...], 0.0)
    h2 = jnp.dot(x1, w2_ref[...], preferred_element_type=jnp.float32)
    out_ref[...] = h2 * ns_ref[:, :1]


def _tc_mid(agg_h, nd, w1, b1_2d, w2p, ns):
    return pl.pallas_call(
        _tc_mid_body,
        grid=(GRID,),
        in_specs=[
            pl.BlockSpec((NC, BLK, FH), lambda i: (0, i, 0)),
            pl.BlockSpec((BLK, OP), lambda i: (i, 0)),
            pl.BlockSpec((F, F), lambda i: (0, 0)),
            pl.BlockSpec((1, F), lambda i: (0, 0)),
            pl.BlockSpec((F, OP), lambda i: (0, 0)),
            pl.BlockSpec((BLK, OP), lambda i: (i, 0)),
        ],
        out_specs=pl.BlockSpec((BLK, OP), lambda i: (i, 0)),
        out_shape=jax.ShapeDtypeStruct((NR, OP), jnp.float32),
    )(agg_h, nd, w1, b1_2d, w2p, ns)


def _tc_final_body(agg_ref, nd_ref, b2_ref, z_ref, att_ref):
    t = (agg_ref[0] + agg_ref[1]) * nd_ref[:, :1] + b2_ref[...]
    x2 = jnp.maximum(t, 0.0)
    zv = jnp.sum(x2, axis=1, keepdims=True) * (1.0 / 5.0)
    z_ref[...] = zv
    att_ref[...] = jnp.ones((BLK, 1), jnp.float32)


def _tc_final(agg2_p, nd, b2p_2d):
    return pl.pallas_call(
        _tc_final_body,
        grid=(GRID,),
        in_specs=[
            pl.BlockSpec((NC, BLK, OP), lambda i: (0, i, 0)),
            pl.BlockSpec((BLK, OP), lambda i: (i, 0)),
            pl.BlockSpec((1, OP), lambda i: (0, 0)),
        ],
        out_specs=[
            pl.BlockSpec((BLK, 1), lambda i: (i, 0)),
            pl.BlockSpec((BLK, 1), lambda i: (i, 0)),
        ],
        out_shape=[
            jax.ShapeDtypeStruct((NR, 1), jnp.float32),
            jax.ShapeDtypeStruct((NR, 1), jnp.float32),
        ],
    )(agg2_p, nd, b2p_2d)


def _pad_edges(idx):
    per = E // NS
    pad = EPT - per
    r = idx.astype(jnp.int32).reshape(NS, per)
    padv = N + (jnp.arange(pad, dtype=jnp.int32) % GR)
    padv = jnp.broadcast_to(padv, (NS, pad))
    return jnp.concatenate([r, padv], axis=1).reshape(NS, NWIN, WIN)


def kernel(features, edge_index, W1, b1, W2, b2, W_att, b_att):
    n_nodes = features.shape[-1]
    x = jnp.reshape(features, (n_nodes, -1))
    x_pad = jnp.pad(x, ((0, NR - N), (0, 0)))

    src = _pad_edges(edge_index[0])
    dst = _pad_edges(edge_index[1])

    ones_w = jnp.ones((WIN, OP), jnp.float32)
    zeros_s16 = jnp.zeros((STR, OP), jnp.float32)
    zeros_s64 = jnp.zeros((STR, FH), jnp.bfloat16)

    x0 = x_pad[:, :FH]
    x1 = x_pad[:, FH:]
    xs0, xs1, ns, nd = _sc_prep(x0, x1, src, dst, ones_w, zeros_s16)
    agg1_h = _sc_agg128(xs0, xs1, src, dst, zeros_s64)

    b1_2d = jnp.reshape(b1, (1, F))
    w2p = jnp.pad(W2, ((0, 0), (0, OP - W2.shape[1])))
    w1p = W1[_PACK_PERM, :]
    h2s = _tc_mid(agg1_h, nd, w1p, b1_2d, w2p, ns)

    agg2_p = _sc_agg16(h2s, src, dst, zeros_s16)

    b2p_2d = jnp.reshape(jnp.pad(b2, (0, OP - b2.shape[0])), (1, OP))
    z2d, att2d = _tc_final(agg2_p, nd, b2p_2d)

    z = z2d[:N, 0]
    att = att2d[:N]
    return (z, att)


# speedup vs baseline: 18.6668x; 1.0275x over previous
import functools

import jax
import jax.numpy as jnp
from jax import lax
from jax.experimental import pallas as pl
from jax.experimental.pallas import tpu as pltpu
from jax.experimental.pallas import tpu_sc as plsc

N = 10000
E = 320000
F = 128
FH = F // 2
OP = 16
NC, NS = 2, 16
WIN = 128
NWIN = 160
NWH = NWIN // 2
EPT = NWIN * WIN
EP = NS * EPT
GR = 240
NR = N + GR
STR = NR // NS
CH1 = 10
CH3 = 8
BLK = 256
GRID = NR // BLK

_PACK_PERM = tuple(
    h * 64 + b * 32 + (p // 2) + 16 * (p % 2)
    for h in range(2) for b in range(2) for p in range(32)
)

_SC_PARAMS = pltpu.CompilerParams(use_tc_tiling_on_sc=False)
_SC_PARAMS_NLP = pltpu.CompilerParams(use_tc_tiling_on_sc=False,
                                      needs_layout_passes=False)


def _sc_mesh():
    return plsc.VectorSubcoreMesh(core_axis_name="c", subcore_axis_name="s")


def _rsqrt16(d):
    h = d * 0.5
    i = plsc.bitcast(d, jnp.int32)
    i = jnp.int32(0x5F3759DF) - (i >> 1)
    y = plsc.bitcast(i, jnp.float32)
    for _ in range(2):
        y = y * (1.5 - h * y * y)
    return y


@functools.partial(
    pl.kernel,
    out_type=(
        jax.ShapeDtypeStruct((NR, FH), jnp.bfloat16),
        jax.ShapeDtypeStruct((NR, FH), jnp.bfloat16),
        jax.ShapeDtypeStruct((NR, OP), jnp.float32),
        jax.ShapeDtypeStruct((NR, OP), jnp.float32),
    ),
    mesh=_sc_mesh(),
    compiler_params=_SC_PARAMS_NLP,
    scratch_types=(
        pltpu.VMEM((NWIN, WIN), jnp.int32),
        pltpu.VMEM((WIN, OP), jnp.float32),
        pltpu.VMEM((STR, OP), jnp.float32),
        pltpu.VMEM((WIN, FH), jnp.float32),
        pltpu.VMEM((WIN, FH), jnp.float32),
        pltpu.VMEM((WIN, FH), jnp.bfloat16),
        pltpu.VMEM((WIN, FH), jnp.bfloat16),
        pltpu.VMEM_SHARED((NR, OP), jnp.float32),
        pltpu.SemaphoreType.DMA,
    ),
)
def _sc_prep(x0_hbm, x1_hbm, src_hbm, dst_hbm, ones_hbm, zeros_hbm,
             xs0_out, xs1_out, ns_out, nd_out,
             idx, ones_v, nsv, xb0, xb1, xbb0, xbb1, sh_deg, sem):
    c = lax.axis_index("c")
    s = lax.axis_index("s")

    @pl.when(c == 0)
    def _():
        pltpu.sync_copy(src_hbm.at[s], idx)

    @pl.when(c == 1)
    def _():
        pltpu.sync_copy(dst_hbm.at[s], idx)

    pltpu.sync_copy(ones_hbm, ones_v)
    pltpu.sync_copy(zeros_hbm, sh_deg.at[pl.ds(s * STR, STR)])
    plsc.subcore_barrier()

    for w in range(CH1):
        pltpu.async_copy(ones_v, sh_deg.at[idx.at[w]], sem, add=True)

    @pl.loop(CH1, NWIN)
    def _(w):
        pltpu.make_async_copy(ones_v, sh_deg.at[idx.at[w - CH1]], sem).wait()
        pltpu.async_copy(ones_v, sh_deg.at[idx.at[w]], sem, add=True)

    @pl.loop(NWIN - CH1, NWIN)
    def _(w):
        pltpu.make_async_copy(ones_v, sh_deg.at[idx.at[w]], sem).wait()

    plsc.subcore_barrier()
    pltpu.sync_copy(sh_deg.at[pl.ds(s * STR, STR)], nsv)

    @pl.loop(0, STR // 2)
    def _(r2):
        r = 2 * r2
        nsv[r] = _rsqrt16(jnp.maximum(nsv[r], 1.0))
        nsv[r + 1] = _rsqrt16(jnp.maximum(nsv[r + 1], 1.0))

    @pl.when(c == 0)
    def _():
        pltpu.sync_copy(nsv, ns_out.at[pl.ds(s * STR, STR)])
        for t in range(STR // WIN):
            base = s * STR + t * WIN
            pltpu.sync_copy(x0_hbm.at[pl.ds(base, WIN)], xb0)
            pltpu.sync_copy(x1_hbm.at[pl.ds(base, WIN)], xb1)

            @pl.loop(0, WIN)
            def _(r):
                nv = nsv[t * WIN + r]
                for src_b, dst_b in ((xb0, xbb0), (xb1, xbb1)):
                    for q in range(FH // 32):
                        a = src_b[r, pl.ds(q * 32, 16)] * nv
                        b = src_b[r, pl.ds(q * 32 + 16, 16)] * nv
                        dst_b[r, pl.ds(q * 32, 32)] = plsc.pack(
                            a, b, format=plsc.PackFormat.INTERLEAVED)

            pltpu.sync_copy(xbb0, xs0_out.at[pl.ds(base, WIN)])
            pltpu.sync_copy(xbb1, xs1_out.at[pl.ds(base, WIN)])

    @pl.when(c == 1)
    def _():
        pltpu.sync_copy(nsv, nd_out.at[pl.ds(s * STR, STR)])


@functools.partial(
    pl.kernel,
    out_type=jax.ShapeDtypeStruct((NC, NR, FH), jnp.bfloat16),
    mesh=_sc_mesh(),
    compiler_params=_SC_PARAMS,
    scratch_types=(
        pltpu.VMEM((NWIN, WIN), jnp.int32),
        pltpu.VMEM((NWIN, WIN), jnp.int32),
        pltpu.VMEM((4, WIN, FH), jnp.bfloat16),
        pltpu.VMEM_SHARED((NR, FH), jnp.bfloat16),
        pltpu.SemaphoreType.DMA,
        pltpu.SemaphoreType.DMA,
    ),
)
def _sc_agg128(xs0_hbm, xs1_hbm, src_hbm, dst_hbm, zeros_hbm, out,
               idx_s, idx_d, rb, sh, gsem, ssem):
    c = lax.axis_index("c")
    s = lax.axis_index("s")
    pltpu.sync_copy(src_hbm.at[s], idx_s)
    pltpu.sync_copy(dst_hbm.at[s], idx_d)
    pltpu.sync_copy(zeros_hbm, sh.at[pl.ds(s * STR, STR)])
    plsc.subcore_barrier()

    def run_half(xs_ref):
        for j in range(4):
            pltpu.async_copy(xs_ref.at[idx_s.at[j]], rb.at[j], gsem)

        @pl.loop(0, NWIN // 4)
        def _(t):
            w0 = 4 * t
            for j in range(4):
                w = w0 + j
                pltpu.make_async_copy(xs_ref.at[idx_s.at[w]],
                                      rb.at[j], gsem).wait()
                pltpu.async_copy(rb.at[j], sh.at[idx_d.at[w]], ssem,
                                 add=True)
            for j in range(4):
                w = w0 + j

                @pl.when(w + 4 < NWIN)
                def _():
                    pltpu.make_async_copy(rb.at[j], sh.at[idx_d.at[w]],
                                          ssem).wait()
                    pltpu.async_copy(xs_ref.at[idx_s.at[w + 4]],
                                     rb.at[j], gsem)

        for j in range(4):
            w = NWIN - 4 + j
            pltpu.make_async_copy(rb.at[j], sh.at[idx_d.at[w]], ssem).wait()

    @pl.when(c == 0)
    def _():
        run_half(xs0_hbm)

    @pl.when(c == 1)
    def _():
        run_half(xs1_hbm)

    plsc.subcore_barrier()
    pltpu.sync_copy(sh.at[pl.ds(s * STR, STR)],
                    out.at[c, pl.ds(s * STR, STR)])


@functools.partial(
    pl.kernel,
    out_type=jax.ShapeDtypeStruct((NC, NR, OP), jnp.float32),
    mesh=_sc_mesh(),
    compiler_params=_SC_PARAMS,
    scratch_types=(
        pltpu.VMEM((NWH, WIN), jnp.int32),
        pltpu.VMEM((NWH, WIN), jnp.int32),
        pltpu.VMEM((CH3, WIN, OP), jnp.float32),
        pltpu.VMEM((CH3, WIN, OP), jnp.float32),
        pltpu.VMEM_SHARED((NR, OP), jnp.float32),
        pltpu.SemaphoreType.DMA,
        pltpu.SemaphoreType.DMA,
    ),
)
def _sc_agg16(rows_hbm, src_hbm, dst_hbm, zeros_hbm, out,
              idx_s, idx_d, r0, r1, sh, sem0, sem1):
    c = lax.axis_index("c")
    s = lax.axis_index("s")
    pltpu.sync_copy(src_hbm.at[s, pl.ds(c * NWH, NWH)], idx_s)
    pltpu.sync_copy(dst_hbm.at[s, pl.ds(c * NWH, NWH)], idx_d)
    pltpu.sync_copy(zeros_hbm, sh.at[pl.ds(s * STR, STR)])
    plsc.subcore_barrier()

    def fire_gathers(k, buf):
        for j in range(CH3):
            pltpu.async_copy(rows_hbm.at[idx_s.at[k * CH3 + j]],
                             buf.at[j], sem0)

    def drain_gathers(k, buf):
        for j in range(CH3):
            pltpu.make_async_copy(rows_hbm.at[idx_s.at[k * CH3 + j]],
                                  buf.at[j], sem0).wait()

    def fire_scatters(k, buf):
        for j in range(CH3):
            pltpu.async_copy(buf.at[j], sh.at[idx_d.at[k * CH3 + j]],
                             sem1, add=True)

    def drain_scatters(k, buf):
        for j in range(CH3):
            pltpu.make_async_copy(buf.at[j], sh.at[idx_d.at[k * CH3 + j]],
                                  sem1).wait()

    ngroups = NWH // CH3
    bufs = (r0, r1)
    fire_gathers(0, r0)
    for k in range(ngroups):
        b = bufs[k % 2]
        drain_gathers(k, b)
        if k + 1 < ngroups:
            fire_gathers(k + 1, bufs[(k + 1) % 2])
        fire_scatters(k, b)
        drain_scatters(k, b)

    plsc.subcore_barrier()
    pltpu.sync_copy(sh.at[pl.ds(s * STR, STR)],
                    out.at[c, pl.ds(s * STR, STR)])


def _tc_mid_body(agg_ref, nd_ref, w1_ref, b1_ref, w2_ref, ns_ref, out_ref):
    t = jnp.concatenate([agg_ref[0], agg_ref[1]], axis=-1)
    t = t.astype(jnp.float32) * nd_ref[:, :1]
    x1 = jnp.dot(t, w1_ref[...], preferred_element_type=jnp.float32)
    x1 = jnp.maximum(x1 + b1_ref[...], 0.0)
    h2 = jnp.dot(x1, w2_ref[...], preferred_element_type=jnp.float32)
    out_ref[...] = h2 * ns_ref[:, :1]


def _tc_mid(agg_h, nd, w1, b1_2d, w2p, ns):
    return pl.pallas_call(
        _tc_mid_body,
        grid=(GRID,),
        in_specs=[
            pl.BlockSpec((NC, BLK, FH), lambda i: (0, i, 0)),
            pl.BlockSpec((BLK, OP), lambda i: (i, 0)),
            pl.BlockSpec((F, F), lambda i: (0, 0)),
            pl.BlockSpec((1, F), lambda i: (0, 0)),
            pl.BlockSpec((F, OP), lambda i: (0, 0)),
            pl.BlockSpec((BLK, OP), lambda i: (i, 0)),
        ],
        out_specs=pl.BlockSpec((BLK, OP), lambda i: (i, 0)),
        out_shape=jax.ShapeDtypeStruct((NR, OP), jnp.float32),
    )(agg_h, nd, w1, b1_2d, w2p, ns)


def _tc_final_body(agg_ref, nd_ref, b2_ref, z_ref, att_ref):
    t = (agg_ref[0] + agg_ref[1]) * nd_ref[:, :1] + b2_ref[...]
    x2 = jnp.maximum(t, 0.0)
    zv = jnp.sum(x2, axis=1, keepdims=True) * (1.0 / 5.0)
    z_ref[...] = zv
    att_ref[...] = jnp.ones((BLK, 1), jnp.float32)


def _tc_final(agg2_p, nd, b2p_2d):
    return pl.pallas_call(
        _tc_final_body,
        grid=(GRID,),
        in_specs=[
            pl.BlockSpec((NC, BLK, OP), lambda i: (0, i, 0)),
            pl.BlockSpec((BLK, OP), lambda i: (i, 0)),
            pl.BlockSpec((1, OP), lambda i: (0, 0)),
        ],
        out_specs=[
            pl.BlockSpec((BLK, 1), lambda i: (i, 0)),
            pl.BlockSpec((BLK, 1), lambda i: (i, 0)),
        ],
        out_shape=[
            jax.ShapeDtypeStruct((NR, 1), jnp.float32),
            jax.ShapeDtypeStruct((NR, 1), jnp.float32),
        ],
    )(agg2_p, nd, b2p_2d)


def _pad_edges(idx):
    per = E // NS
    pad = EPT - per
    r = idx.astype(jnp.int32).reshape(NS, per)
    padv = N + (jnp.arange(pad, dtype=jnp.int32) % GR)
    padv = jnp.broadcast_to(padv, (NS, pad))
    return jnp.concatenate([r, padv], axis=1).reshape(NS, NWIN, WIN)


def kernel(features, edge_index, W1, b1, W2, b2, W_att, b_att):
    n_nodes = features.shape[-1]
    x = jnp.reshape(features, (n_nodes, -1))
    x_pad = jnp.pad(x, ((0, NR - N), (0, 0)))

    src = _pad_edges(edge_index[0])
    dst = _pad_edges(edge_index[1])

    ones_w = jnp.ones((WIN, OP), jnp.float32)
    zeros_s16 = jnp.zeros((STR, OP), jnp.float32)
    zeros_s64 = jnp.zeros((STR, FH), jnp.bfloat16)

    x0 = x_pad[:, :FH]
    x1 = x_pad[:, FH:]
    xs0, xs1, ns, nd = _sc_prep(x0, x1, src, dst, ones_w, zeros_s16)
    agg1_h = _sc_agg128(xs0, xs1, src, dst, zeros_s64)

    b1_2d = jnp.reshape(b1, (1, F))
    w2p = jnp.pad(W2, ((0, 0), (0, OP - W2.shape[1])))
    w1p = W1[_PACK_PERM, :]
    h2s = _tc_mid(agg1_h, nd, w1p, b1_2d, w2p, ns)

    agg2_p = _sc_agg16(h2s, src, dst, zeros_s16)

    b2p_2d = jnp.reshape(jnp.pad(b2, (0, OP - b2.shape[0])), (1, OP))
    z2d, att2d = _tc_final(agg2_p, nd, b2p_2d)

    z = z2d[:N, 0]
    att = att2d[:N]
    return (z, att)
